# baseline (speedup 1.0000x reference)
import functools

import jax
import jax.numpy as jnp
from jax import lax
from jax.experimental import pallas as pl
from jax.experimental.pallas import tpu as pltpu
from jax.experimental.pallas import tpu_sc as plsc

_K = 64
_N = 8192
_Q = 8
_B = 32
_ROWS = _B * _Q
_NW = 32
_RPW = _ROWS // _NW
_G = 64
_GSZ = _N // _G
_GV = _GSZ // 16


_GDN = lax.GatherDimensionNumbers(
    offset_dims=(), collapsed_slice_dims=(0,), start_index_map=(0,))


def _shuf(v, idx):
    return lax.gather(v, idx[:, None], _GDN, (1,),
                      mode=lax.GatherScatterMode.PROMISE_IN_BOUNDS)


def _sc_body(x_hbm, thr_hbm, feat_hbm, idx_hbm, v0, v1, v2, v3, f0, f1, f2,
             f3, oidx_v, thr_v, si0, si1, si2, si3, so0, so1, so2, so3):
    wid = lax.axis_index("s") * 2 + lax.axis_index("c")
    lane = lax.iota(jnp.int32, 16)
    lane0 = lane == 0
    ones16 = jnp.full((16,), 1.0, jnp.float32)
    zeros16 = jnp.zeros((16,), jnp.float32)
    neg16 = jnp.full((16,), -1.0, jnp.float32)
    xors = [lane ^ 1, lane ^ 2, lane ^ 4, lane ^ 8]
    lane15 = jnp.full((16,), 15, jnp.int32)
    vals = [v0, v1, v2, v3]
    feat = [f0, f1, f2, f3]
    sin = [si0, si1, si2, si3]
    sout = [so0, so1, so2, so3]

    def vmax_all(v):
        for xi in xors:
            v = jnp.maximum(v, _shuf(v, xi))
        return v

    def vmin_all(v):
        for xi in xors:
            v = jnp.minimum(v, _shuf(v, xi))
        return v

    pltpu.sync_copy(thr_hbm, thr_v)
    thrv = thr_v[pl.ds(0, 16)]

    def zf(i, c):
        for f in feat:
            f[pl.ds(i * 16, 16)] = zeros16
        return c

    lax.fori_loop(0, _N // 16, zf, 0)

    row0 = wid * _RPW
    npairs = _RPW // 2
    in_h = [None] * 4
    out_h = [None] * 4

    def start_in(jj):
        s0, s1 = 2 * (jj % 2), 2 * (jj % 2) + 1
        in_h[s0] = pltpu.async_copy(x_hbm.at[row0 + 2 * jj], vals[s0],
                                    sin[s0])
        in_h[s1] = pltpu.async_copy(x_hbm.at[row0 + 2 * jj + 1], vals[s1],
                                    sin[s1])

    start_in(0)
    for jj in range(npairs):
        s0, s1 = 2 * (jj % 2), 2 * (jj % 2) + 1
        ja = jj * 2
        jb = ja + 1
        vals_a, vals_b = vals[s0], vals[s1]
        feat_a, feat_b = feat[s0], feat[s1]
        in_h[s0].wait()
        in_h[s1].wait()
        if jj + 1 < npairs:
            start_in(jj + 1)

        def build(g, gc, vals_a=vals_a, vals_b=vals_b):
            base = g * _GSZ
            gmod = lane == (g & 15)
            gdiv = g // 16
            out = list(gc)
            for which, vref in ((0, vals_a), (1, vals_b)):
                gm = None
                for t in range(_GV):
                    v = vref[pl.ds(base + t * 16, 16)]
                    v = jnp.where(v >= thrv, v, zeros16)
                    vref[pl.ds(base + t * 16, 16)] = v
                    gm = v if gm is None else jnp.maximum(gm, v)
                gmv = _shuf(plsc.cummax(gm), lane15)
                for xq in range(4):
                    k = which * 4 + xq
                    out[k] = jnp.where(gmod & (gdiv == xq), gmv, out[k])
            return tuple(out)

        gcar = lax.fori_loop(0, _G, build, (zeros16,) * 8)

        if out_h[s0] is not None:
            out_h[s0].wait()
            out_h[s1].wait()
            for jold, fref in ((ja - 4, feat_a), (jb - 4, feat_b)):
                for t in range(_K // 16):
                    idxv = oidx_v[pl.ds(jold * _K + t * 16, 16)]
                    plsc.store_scatter(fref, [idxv], zeros16)

        def ext(r, gc, vals_a=vals_a, vals_b=vals_b, feat_a=feat_a,
                feat_b=feat_b, ja=ja, jb=jb):
            out = list(gc)
            for which, vref, fref, jrow in ((0, vals_a, feat_a, ja),
                                            (1, vals_b, feat_b, jb)):
                g0, g1, g2, g3 = out[which * 4:which * 4 + 4]
                mm = jnp.maximum(jnp.maximum(g0, g1), jnp.maximum(g2, g3))
                big = vmax_all(mm)
                c0 = jnp.where(g0 == big, lane, 64)
                c1 = jnp.where(g1 == big, lane + 16, 64)
                c2 = jnp.where(g2 == big, lane + 32, 64)
                c3 = jnp.where(g3 == big, lane + 48, 64)
                gsv = vmin_all(jnp.minimum(jnp.minimum(c0, c1),
                                           jnp.minimum(c2, c3)))
                basev = gsv * _GSZ
                pc = None
                ivs = []
                vts = []
                for t in range(_GV):
                    iv = basev + lane + t * 16
                    v = plsc.load_gather(vref, [iv])
                    ivs.append(iv)
                    vts.append(v)
                    cd = jnp.where(v == big, iv, _N)
                    pc = cd if pc is None else jnp.minimum(pc, cd)
                pvec = vmin_all(pc)
                plsc.store_scatter(oidx_v, [jnp.full((16,), jrow * _K,
                                            jnp.int32) + r], pvec,
                                   mask=lane0)
                plsc.store_scatter(vref, [pvec], neg16, mask=lane0)
                gm = None
                for iv, v in zip(ivs, vts):
                    vk = jnp.where(iv == pvec, neg16, v)
                    gm = vk if gm is None else jnp.maximum(gm, vk)
                gmv = vmax_all(gm)
                gmod = lane == (gsv & 15)
                gdiv = gsv >> 4
                for xq in range(4):
                    k = which * 4 + xq
                    out[k] = jnp.where(gmod & (gdiv == xq), gmv, out[k])
            return tuple(out)

        lax.fori_loop(0, _K, ext, gcar)

        for jrow, fref in ((ja, feat_a), (jb, feat_b)):
            for t in range(_K // 16):
                idxv = oidx_v[pl.ds(jrow * _K + t * 16, 16)]
                plsc.store_scatter(fref, [idxv], ones16)

        out_h[s0] = pltpu.async_copy(feat_a, feat_hbm.at[row0 + ja],
                                     sout[s0])
        out_h[s1] = pltpu.async_copy(feat_b, feat_hbm.at[row0 + jb],
                                     sout[s1])

    for h in out_h:
        if h is not None:
            h.wait()
    pltpu.sync_copy(oidx_v, idx_hbm.at[pl.ds(wid * _RPW * _K, _RPW * _K)])


@jax.jit
def _sc_call(x2, thr16):
    mesh = plsc.VectorSubcoreMesh(core_axis_name="c", subcore_axis_name="s")
    f = functools.partial(
        pl.kernel,
        out_type=[
            jax.ShapeDtypeStruct((_ROWS, _N), jnp.float32),
            jax.ShapeDtypeStruct((_ROWS * _K,), jnp.int32),
        ],
        mesh=mesh,
        scratch_types=(
            [pltpu.VMEM((_N,), jnp.float32)] * 8
            + [pltpu.VMEM((_RPW * _K,), jnp.int32),
               pltpu.VMEM((128,), jnp.float32)]
            + [pltpu.SemaphoreType.DMA] * 8
        ),
        compiler_params=pltpu.CompilerParams(needs_layout_passes=False),
    )(_sc_body)
    return f(x2, thr16)


def _tc_valid_body(thr_ref, x_ref, valid_ref):
    valid_ref[...] = (x_ref[...] >= thr_ref[0]).astype(jnp.int8)


def kernel(x, threshold):
    B, Q, N = x.shape
    thr16 = jnp.broadcast_to(threshold.astype(jnp.float32), (128,))
    x2 = x.reshape(B * Q, N)
    feat2, inds1 = _sc_call(x2, thr16)
    inds2 = inds1.reshape(B * Q, _K)
    valid = pl.pallas_call(
        _tc_valid_body,
        grid=(B,),
        in_specs=[
            pl.BlockSpec(memory_space=pltpu.SMEM),
            pl.BlockSpec((1, Q, N), lambda b: (b, 0, 0)),
        ],
        out_specs=pl.BlockSpec((1, Q, N), lambda b: (b, 0, 0)),
        out_shape=jax.ShapeDtypeStruct((B, Q, N), jnp.int8),
    )(jnp.reshape(threshold.astype(jnp.float32), (1,)), x)
    return (feat2.reshape(B, Q, N), inds2.reshape(B, Q, _K),
            valid.astype(bool))


# speedup vs baseline: 1.0088x; 1.0088x over previous
import functools

import jax
import jax.numpy as jnp
from jax import lax
from jax.experimental import pallas as pl
from jax.experimental.pallas import tpu as pltpu
from jax.experimental.pallas import tpu_sc as plsc

_K = 64
_N = 8192
_Q = 8
_B = 32
_ROWS = _B * _Q
_NW = 32
_RPW = _ROWS // _NW
_G = 64
_GSZ = _N // _G
_GV = _GSZ // 16


_GDN = lax.GatherDimensionNumbers(
    offset_dims=(), collapsed_slice_dims=(0,), start_index_map=(0,))


def _shuf(v, idx):
    return lax.gather(v, idx[:, None], _GDN, (1,),
                      mode=lax.GatherScatterMode.PROMISE_IN_BOUNDS)


def _sc_body(x_hbm, thr_hbm, feat_hbm, idx_hbm, v0, v1, v2, v3, f0, f1, f2,
             f3, oidx_v, thr_v, si0, si1, si2, si3, so0, so1, so2, so3):
    wid = lax.axis_index("s") * 2 + lax.axis_index("c")
    lane = lax.iota(jnp.int32, 16)
    lane0 = lane == 0
    ones16 = jnp.full((16,), 1.0, jnp.float32)
    zeros16 = jnp.zeros((16,), jnp.float32)
    neg16 = jnp.full((16,), -1.0, jnp.float32)
    xors = [lane ^ 1, lane ^ 2, lane ^ 4, lane ^ 8]
    lane15 = jnp.full((16,), 15, jnp.int32)
    vals = [v0, v1, v2, v3]
    feat = [f0, f1, f2, f3]
    sin = [si0, si1, si2, si3]
    sout = [so0, so1, so2, so3]

    def vmax_all(v):
        for xi in xors:
            v = jnp.maximum(v, _shuf(v, xi))
        return v

    def vmin_all(v):
        for xi in xors:
            v = jnp.minimum(v, _shuf(v, xi))
        return v

    pltpu.sync_copy(thr_hbm, thr_v)
    thrv = thr_v[pl.ds(0, 16)]

    def zf(i, c):
        for f in feat:
            f[pl.ds(i * 16, 16)] = zeros16
        return c

    lax.fori_loop(0, _N // 16, zf, 0)

    row0 = wid * _RPW
    npairs = _RPW // 2
    in_h = [None] * 4
    out_h = [None] * 4

    def start_in(jj):
        s0, s1 = 2 * (jj % 2), 2 * (jj % 2) + 1
        in_h[s0] = pltpu.async_copy(x_hbm.at[row0 + 2 * jj], vals[s0],
                                    sin[s0])
        in_h[s1] = pltpu.async_copy(x_hbm.at[row0 + 2 * jj + 1], vals[s1],
                                    sin[s1])

    start_in(0)
    for jj in range(npairs):
        s0, s1 = 2 * (jj % 2), 2 * (jj % 2) + 1
        ja = jj * 2
        jb = ja + 1
        vals_a, vals_b = vals[s0], vals[s1]
        feat_a, feat_b = feat[s0], feat[s1]
        in_h[s0].wait()
        in_h[s1].wait()
        if jj + 1 < npairs:
            start_in(jj + 1)

        def build(g, gc, vals_a=vals_a, vals_b=vals_b):
            base = g * _GSZ
            gmod = lane == (g & 15)
            gdiv = g // 16
            out = list(gc)
            for which, vref in ((0, vals_a), (1, vals_b)):
                gm = None
                for t in range(_GV):
                    v = vref[pl.ds(base + t * 16, 16)]
                    v = jnp.where(v >= thrv, v, zeros16)
                    vref[pl.ds(base + t * 16, 16)] = v
                    gm = v if gm is None else jnp.maximum(gm, v)
                gmv = _shuf(plsc.cummax(gm), lane15)
                for xq in range(4):
                    k = which * 4 + xq
                    out[k] = jnp.where(gmod & (gdiv == xq), gmv, out[k])
            return tuple(out)

        gcar = lax.fori_loop(0, _G, build, (zeros16,) * 8)

        def find(g0, g1, g2, g3):
            mm = jnp.maximum(jnp.maximum(g0, g1), jnp.maximum(g2, g3))
            big = vmax_all(mm)
            c0 = jnp.where(g0 == big, lane, 64)
            c1 = jnp.where(g1 == big, lane + 16, 64)
            c2 = jnp.where(g2 == big, lane + 32, 64)
            c3 = jnp.where(g3 == big, lane + 48, 64)
            gsv = vmin_all(jnp.minimum(jnp.minimum(c0, c1),
                                       jnp.minimum(c2, c3)))
            return big, gsv

        biga, gsva = find(*gcar[0:4])
        bigb, gsvb = find(*gcar[4:8])
        gcar = tuple(gcar) + (biga, gsva, bigb, gsvb)

        if out_h[s0] is not None:
            out_h[s0].wait()
            out_h[s1].wait()
            for jold, fref in ((ja - 4, feat_a), (jb - 4, feat_b)):
                for t in range(_K // 16):
                    idxv = oidx_v[pl.ds(jold * _K + t * 16, 16)]
                    plsc.store_scatter(fref, [idxv], zeros16)

        def ext(r, gc, vals_a=vals_a, vals_b=vals_b, ja=ja, jb=jb):
            out = list(gc[:8])
            extras = list(gc[8:])
            for which, vref, jrow in ((0, vals_a, ja), (1, vals_b, jb)):
                big = extras[2 * which]
                gsv = extras[2 * which + 1]
                gx = out[which * 4:which * 4 + 4]
                basev = gsv * _GSZ
                pc = None
                ivs = []
                vts = []
                for t in range(_GV):
                    iv = basev + lane + t * 16
                    v = plsc.load_gather(vref, [iv])
                    ivs.append(iv)
                    vts.append(v)
                    cd = jnp.where(v == big, iv, _N)
                    pc = cd if pc is None else jnp.minimum(pc, cd)
                pvec = vmin_all(pc)
                plsc.store_scatter(oidx_v, [jnp.full((16,), jrow * _K,
                                            jnp.int32) + r], pvec,
                                   mask=lane0)
                plsc.store_scatter(vref, [pvec], neg16, mask=lane0)
                gm = None
                for iv, v in zip(ivs, vts):
                    vk = jnp.where(iv == pvec, neg16, v)
                    gm = vk if gm is None else jnp.maximum(gm, vk)
                gmv = vmax_all(gm)
                gmod = lane == (gsv & 15)
                gdiv = gsv >> 4
                wm = [gmod & (gdiv == xq) for xq in range(4)]
                gxm = [jnp.where(wm[xq], -2.0, gx[xq]) for xq in range(4)]
                m2 = vmax_all(jnp.maximum(jnp.maximum(gxm[0], gxm[1]),
                                          jnp.maximum(gxm[2], gxm[3])))
                bign = jnp.maximum(m2, gmv)
                co = None
                for xq in range(4):
                    cq = jnp.where(gxm[xq] == bign, lane + 16 * xq, 64)
                    co = cq if co is None else jnp.minimum(co, cq)
                gsvn = jnp.minimum(vmin_all(co),
                                   jnp.where(gmv == bign, gsv, 64))
                for xq in range(4):
                    out[which * 4 + xq] = jnp.where(wm[xq], gmv,
                                                    out[which * 4 + xq])
                extras[2 * which] = bign
                extras[2 * which + 1] = gsvn
            return tuple(out) + tuple(extras)

        lax.fori_loop(0, _K, ext, gcar)

        for jrow, fref in ((ja, feat_a), (jb, feat_b)):
            for t in range(_K // 16):
                idxv = oidx_v[pl.ds(jrow * _K + t * 16, 16)]
                plsc.store_scatter(fref, [idxv], ones16)

        out_h[s0] = pltpu.async_copy(feat_a, feat_hbm.at[row0 + ja],
                                     sout[s0])
        out_h[s1] = pltpu.async_copy(feat_b, feat_hbm.at[row0 + jb],
                                     sout[s1])

    for h in out_h:
        if h is not None:
            h.wait()
    pltpu.sync_copy(oidx_v, idx_hbm.at[pl.ds(wid * _RPW * _K, _RPW * _K)])


@jax.jit
def _sc_call(x2, thr16):
    mesh = plsc.VectorSubcoreMesh(core_axis_name="c", subcore_axis_name="s")
    f = functools.partial(
        pl.kernel,
        out_type=[
            jax.ShapeDtypeStruct((_ROWS, _N), jnp.float32),
            jax.ShapeDtypeStruct((_ROWS * _K,), jnp.int32),
        ],
        mesh=mesh,
        scratch_types=(
            [pltpu.VMEM((_N,), jnp.float32)] * 8
            + [pltpu.VMEM((_RPW * _K,), jnp.int32),
               pltpu.VMEM((128,), jnp.float32)]
            + [pltpu.SemaphoreType.DMA] * 8
        ),
        compiler_params=pltpu.CompilerParams(needs_layout_passes=False),
    )(_sc_body)
    return f(x2, thr16)


def _tc_valid_body(thr_ref, x_ref, valid_ref):
    valid_ref[...] = (x_ref[...] >= thr_ref[0]).astype(jnp.int8)


def kernel(x, threshold):
    B, Q, N = x.shape
    thr16 = jnp.broadcast_to(threshold.astype(jnp.float32), (128,))
    x2 = x.reshape(B * Q, N)
    feat2, inds1 = _sc_call(x2, thr16)
    inds2 = inds1.reshape(B * Q, _K)
    valid = pl.pallas_call(
        _tc_valid_body,
        grid=(B,),
        in_specs=[
            pl.BlockSpec(memory_space=pltpu.SMEM),
            pl.BlockSpec((1, Q, N), lambda b: (b, 0, 0)),
        ],
        out_specs=pl.BlockSpec((1, Q, N), lambda b: (b, 0, 0)),
        out_shape=jax.ShapeDtypeStruct((B, Q, N), jnp.int8),
    )(jnp.reshape(threshold.astype(jnp.float32), (1,)), x)
    return (feat2.reshape(B, Q, N), inds2.reshape(B, Q, _K),
            valid.astype(bool))
